# speedup vs baseline: 11.0889x; 11.0889x over previous
import functools

import jax
import jax.numpy as jnp
from jax import lax
from jax.experimental import pallas as pl
from jax.experimental.pallas import tpu as pltpu
from jax.experimental.pallas import tpu_sc as plsc

NNODE = 10000
F = 128
NP = 10240
NC, NS = 2, 16
NW = NC * NS
K = 128
CH = 79
EDGES = 320000
EP = NW * CH * K
RB = NP // NS
RB2 = 2048
GRID = NP // RB2

_mesh = plsc.VectorSubcoreMesh(
    core_axis_name="c", subcore_axis_name="s", num_cores=NC, num_subcores=NS)


def _deg_body(col_hbm, zeros_hbm, out_hbm, cidx, ones_v, degsh):
    c = lax.axis_index("c")
    s = lax.axis_index("s")
    wid = c * NS + s
    pltpu.sync_copy(zeros_hbm.at[pl.ds(s * RB, RB)], degsh.at[pl.ds(s * RB, RB)])
    for i in range(K // 16):
        ones_v[pl.ds(i * 16, 16)] = jnp.ones((16,), jnp.float32)
    plsc.subcore_barrier()
    base = wid * (CH * K)

    def body(j, carry):
        off = pl.multiple_of(base + j * K, K)
        pltpu.sync_copy(col_hbm.at[pl.ds(off, K)], cidx)
        pltpu.sync_copy(ones_v, degsh.at[cidx], add=True)
        return carry

    lax.fori_loop(0, CH, body, 0)
    plsc.subcore_barrier()
    pltpu.sync_copy(degsh.at[pl.ds(s * RB, RB)], out_hbm.at[c, pl.ds(s * RB, RB)])


_deg_kernel = pl.kernel(
    _deg_body,
    out_type=jax.ShapeDtypeStruct((NC, NP), jnp.float32),
    mesh=_mesh,
    scratch_types=[
        pltpu.VMEM((K,), jnp.int32),
        pltpu.VMEM((K,), jnp.float32),
        pltpu.VMEM_SHARED((NP,), jnp.float32),
    ],
)


def _prop_body(row_hbm, col_hbm, u_hbm, zeros_hbm, out_hbm,
               ridx, cidx, rows, accsh, sem):
    c = lax.axis_index("c")
    s = lax.axis_index("s")
    wid = c * NS + s
    pltpu.sync_copy(zeros_hbm.at[pl.ds(s * RB, RB)], accsh.at[pl.ds(s * RB, RB)])
    plsc.subcore_barrier()
    base = wid * (CH * K)

    def body(j, carry):
        off = pl.multiple_of(base + j * K, K)
        pltpu.sync_copy(row_hbm.at[pl.ds(off, K)], ridx)
        pltpu.sync_copy(col_hbm.at[pl.ds(off, K)], cidx)
        pltpu.async_copy(u_hbm.at[ridx], rows, sem).wait()
        pltpu.sync_copy(rows, accsh.at[cidx], add=True)
        return carry

    lax.fori_loop(0, CH, body, 0)
    plsc.subcore_barrier()
    pltpu.sync_copy(accsh.at[pl.ds(s * RB, RB)], out_hbm.at[c, pl.ds(s * RB, RB)])


_prop_kernel = pl.kernel(
    _prop_body,
    out_type=jax.ShapeDtypeStruct((NC, NP, F), jnp.float32),
    mesh=_mesh,
    scratch_types=[
        pltpu.VMEM((K,), jnp.int32),
        pltpu.VMEM((K,), jnp.int32),
        pltpu.VMEM((K, F), jnp.float32),
        pltpu.VMEM_SHARED((NP, F), jnp.float32),
        pltpu.SemaphoreType.DMA,
    ],
)


def _scale_body(dt_ref, x_ref, dinv_ref, u_ref):
    d = dt_ref[:, 0:1] + dt_ref[:, 1:2] + 1.0
    dinv = lax.rsqrt(d)
    dinvb = jnp.broadcast_to(dinv, (RB2, F))
    dinv_ref[...] = dinvb
    u_ref[...] = x_ref[...] * dinvb


_scale_kernel = pl.pallas_call(
    _scale_body,
    grid=(GRID,),
    in_specs=[
        pl.BlockSpec((RB2, NC), lambda i: (i, 0)),
        pl.BlockSpec((RB2, F), lambda i: (i, 0)),
    ],
    out_specs=[
        pl.BlockSpec((RB2, F), lambda i: (i, 0)),
        pl.BlockSpec((RB2, F), lambda i: (i, 0)),
    ],
    out_shape=[
        jax.ShapeDtypeStruct((NP, F), jnp.float32),
        jax.ShapeDtypeStruct((NP, F), jnp.float32),
    ],
)

_DOT_T = (((1,), (1,)), ((), ()))


def _mm1_body(acc_ref, u_ref, dinv_ref, w_ref, b_ref, u2_ref):
    sagg = acc_ref[0] + acc_ref[1] + u_ref[...]
    agg = sagg * dinv_ref[...]
    h = lax.dot_general(agg, w_ref[...], _DOT_T,
                        preferred_element_type=jnp.float32) + b_ref[...]
    u2_ref[...] = jnp.maximum(h, 0.0) * dinv_ref[...]


_mm1_kernel = pl.pallas_call(
    _mm1_body,
    grid=(GRID,),
    in_specs=[
        pl.BlockSpec((NC, RB2, F), lambda i: (0, i, 0)),
        pl.BlockSpec((RB2, F), lambda i: (i, 0)),
        pl.BlockSpec((RB2, F), lambda i: (i, 0)),
        pl.BlockSpec((F, F), lambda i: (0, 0)),
        pl.BlockSpec((1, F), lambda i: (0, 0)),
    ],
    out_specs=pl.BlockSpec((RB2, F), lambda i: (i, 0)),
    out_shape=jax.ShapeDtypeStruct((NP, F), jnp.float32),
)


def _mm2_body(acc_ref, u_ref, dinv_ref, wmu_ref, bmu_ref, wls_ref, bls_ref,
              mu_ref, ls_ref):
    sagg = acc_ref[0] + acc_ref[1] + u_ref[...]
    agg = sagg * dinv_ref[...]
    mu_ref[...] = lax.dot_general(agg, wmu_ref[...], _DOT_T,
                                  preferred_element_type=jnp.float32) + bmu_ref[...]
    ls_ref[...] = lax.dot_general(agg, wls_ref[...], _DOT_T,
                                  preferred_element_type=jnp.float32) + bls_ref[...]


_mm2_kernel = pl.pallas_call(
    _mm2_body,
    grid=(GRID,),
    in_specs=[
        pl.BlockSpec((NC, RB2, F), lambda i: (0, i, 0)),
        pl.BlockSpec((RB2, F), lambda i: (i, 0)),
        pl.BlockSpec((RB2, F), lambda i: (i, 0)),
        pl.BlockSpec((F, F), lambda i: (0, 0)),
        pl.BlockSpec((1, F), lambda i: (0, 0)),
        pl.BlockSpec((F, F), lambda i: (0, 0)),
        pl.BlockSpec((1, F), lambda i: (0, 0)),
    ],
    out_specs=[
        pl.BlockSpec((RB2, F), lambda i: (i, 0)),
        pl.BlockSpec((RB2, F), lambda i: (i, 0)),
    ],
    out_shape=[
        jax.ShapeDtypeStruct((NP, F), jnp.float32),
        jax.ShapeDtypeStruct((NP, F), jnp.float32),
    ],
)


def kernel(x, edge_index, W1, b1, Wmu, bmu, Wls, bls):
    pad = EP - EDGES
    fill = jnp.full((pad,), NNODE, jnp.int32)
    rowp = jnp.concatenate([edge_index[0], fill])
    colp = jnp.concatenate([edge_index[1], fill])
    xp = jnp.pad(x, ((0, NP - NNODE), (0, 0)))
    z1 = jnp.zeros((NP,), jnp.float32)
    z2 = jnp.zeros((NP, F), jnp.float32)

    degp = _deg_kernel(colp, z1)
    degt = degp.T
    dinvb, u1 = _scale_kernel(degt, xp)

    acc1 = _prop_kernel(rowp, colp, u1, z2)
    u2 = _mm1_kernel(acc1, u1, dinvb, W1, b1.reshape(1, F))

    acc2 = _prop_kernel(rowp, colp, u2, z2)
    mu, ls = _mm2_kernel(acc2, u2, dinvb, Wmu, bmu.reshape(1, F),
                         Wls, bls.reshape(1, F))
    return mu[:NNODE], ls[:NNODE]
